# Initial kernel scaffold; baseline (speedup 1.0000x reference)
import jax
import jax.numpy as jnp
from jax.experimental import pallas as pl


def kernel(nf, ef, gf, params, edge_index):
    raise NotImplementedError("write your pallas kernel here")


# speedup vs baseline: 1.3100x; 1.3100x over previous
import functools
from functools import partial

import jax
import jax.numpy as jnp
from jax import lax
from jax.experimental import pallas as pl
from jax.experimental.pallas import tpu as pltpu

N_NODES_C = 10000
N_EDGES_C = 320000
BE = 2560
NB = N_EDGES_C // BE
EH_C = 32
HID = 64
BF = jnp.bfloat16
F32 = jnp.float32


def _dot(a, b):
    return jnp.dot(a.astype(BF), b.astype(BF), preferred_element_type=F32)


def _ef_stats_body(ef_ref, out_ref):
    x = ef_ref[...]
    s = jnp.sum(x, axis=0)
    ss = jnp.sum(x * x, axis=0)
    out_ref[0, 0, :] = jnp.concatenate([s, ss])


def _ef_stats(ef):
    E, F = ef.shape
    return pl.pallas_call(
        _ef_stats_body,
        grid=(NB,),
        in_specs=[pl.BlockSpec((BE, F), lambda i: (i, 0))],
        out_specs=pl.BlockSpec((1, 1, 2 * F), lambda i: (i, 0, 0)),
        out_shape=jax.ShapeDtypeStruct((NB, 1, 2 * F), jnp.float32),
    )(ef)


def _edge_prep_body(ei, ni, nf_ref, degs_ref, degd_ref, part_ref,
                    gamma_ref, beta_ref, w1_ref, b1_ref,
                    a_ref, b_ref, bias_ref, mv_ref):
    E = float(N_EDGES_C)
    nf = nf_ref[...]
    part = part_ref[...]
    ef_mu = jnp.sum(part[:, 0, :ei], axis=0)[None, :] / E
    ef_ms = jnp.sum(part[:, 0, ei:2 * ei], axis=0)[None, :] / E
    degs = degs_ref[...]
    degd = degd_ref[...]
    s_mu = jnp.sum(nf * degs, axis=0)[None, :] / E
    s_ms = jnp.sum(nf * nf * degs, axis=0)[None, :] / E
    d_mu = jnp.sum(nf * degd, axis=0)[None, :] / E
    d_ms = jnp.sum(nf * nf * degd, axis=0)[None, :] / E
    K = ei + 2 * ni
    gamma = gamma_ref[...]
    beta = beta_ref[...]
    w1 = w1_ref[...]
    sg_s = jnp.sqrt((s_ms - s_mu * s_mu) + 1e-5)
    sg_d = jnp.sqrt((d_ms - d_mu * d_mu) + 1e-5)
    xs = (nf - s_mu) / sg_s * gamma[:, ei:ei + ni] + beta[:, ei:ei + ni]
    xd = (nf - d_mu) / sg_d * gamma[:, ei + ni:K] + beta[:, ei + ni:K]
    a_ref[...] = _dot(xs, w1[ei:ei + ni, :])
    b_ref[...] = _dot(xd, w1[ei + ni:K, :])
    bias_ref[...] = b1_ref[...] + _dot(beta[:, K:], w1[K:, :])
    sg_ef = jnp.sqrt((ef_ms - ef_mu * ef_mu) + 1e-5)
    mv_ref[...] = jnp.concatenate([ef_mu, sg_ef], axis=1)


def _edge_prep(nf, degs, degd, part, em, ei):
    N, ni = nf.shape
    return pl.pallas_call(
        partial(_edge_prep_body, ei, ni),
        out_shape=[
            jax.ShapeDtypeStruct((N, HID), jnp.float32),
            jax.ShapeDtypeStruct((N, HID), jnp.float32),
            jax.ShapeDtypeStruct((1, HID), jnp.float32),
            jax.ShapeDtypeStruct((1, 2 * ei), jnp.float32),
        ],
    )(nf, degs, degd, part, em["gamma"][None, :], em["beta"][None, :],
      em["W1"], em["b1"][None, :])


def _edge_body(residual, with_att, ei, g_ref, ef_ref, w1e_ref, gb_ref,
               mv_ref, bias_ref, w2_ref, b2_ref, att_ref,
               uef_ref, stat_ref, s_ref):
    g = g_ref[...]
    ef = ef_ref[...]
    mu = mv_ref[0, :ei][None, :]
    sg = mv_ref[0, ei:][None, :]
    gamma = gb_ref[0, :ei][None, :]
    beta = gb_ref[0, ei:][None, :]
    xe = (ef - mu) / sg * gamma + beta
    pre = g + _dot(xe, w1e_ref[...]) + bias_ref[...]
    h = jnp.maximum(pre, 0.0)
    uef = _dot(h, w2_ref[...]) + b2_ref[...]
    if residual:
        uef = uef + ef
    uef_ref[...] = uef
    s = jnp.sum(uef, axis=0)
    ss = jnp.sum(uef * uef, axis=0)
    if with_att:
        sa = _dot(uef, att_ref[...])
        s_ref[...] = sa
        mx = jnp.full((8,), jnp.max(sa), jnp.float32)
    else:
        mx = jnp.zeros((8,), jnp.float32)
    stat_ref[0, 0, :] = jnp.concatenate([s, ss, mx])


def _edge_mlp(G, ef, mv, bias, em, att, residual, with_att):
    E, _ = G.shape
    ei = ef.shape[1]
    att2 = att[:, None] if att is not None else jnp.zeros((EH_C, 1), jnp.float32)
    gb = jnp.concatenate([em["gamma"][None, :ei], em["beta"][None, :ei]],
                         axis=1)
    outs = pl.pallas_call(
        partial(_edge_body, residual, with_att, ei),
        grid=(NB,),
        in_specs=[
            pl.BlockSpec((BE, HID), lambda i: (i, 0)),
            pl.BlockSpec((BE, ei), lambda i: (i, 0)),
            pl.BlockSpec((ei, HID), lambda i: (0, 0)),
            pl.BlockSpec((1, 2 * ei), lambda i: (0, 0)),
            pl.BlockSpec((1, 2 * ei), lambda i: (0, 0)),
            pl.BlockSpec((1, HID), lambda i: (0, 0)),
            pl.BlockSpec((HID, EH_C), lambda i: (0, 0)),
            pl.BlockSpec((1, EH_C), lambda i: (0, 0)),
            pl.BlockSpec((EH_C, 1), lambda i: (0, 0)),
        ],
        out_specs=[
            pl.BlockSpec((BE, EH_C), lambda i: (i, 0)),
            pl.BlockSpec((1, 1, 72), lambda i: (i, 0, 0)),
            pl.BlockSpec((BE, 1), lambda i: (i, 0)),
        ],
        out_shape=[
            jax.ShapeDtypeStruct((E, EH_C), jnp.float32),
            jax.ShapeDtypeStruct((NB, 1, 72), jnp.float32),
            jax.ShapeDtypeStruct((E, 1), jnp.float32),
        ],
    )(G, ef, em["W1"][:ei], gb, mv, bias, em["W2"], em["b2"][None, :], att2)
    return outs


def _att_exp_body(uef_ref, s_ref, stat_ref, ut_ref):
    m = jnp.max(stat_ref[:, 0, 64:72])
    t = jnp.exp(s_ref[...] - m)
    ut_ref[:, :EH_C] = uef_ref[...] * t
    ut_ref[:, EH_C:] = jnp.concatenate(
        [t, jnp.zeros((t.shape[0], 15), jnp.float32)], axis=1)


def _att_exp(uef, s, stats):
    E = uef.shape[0]
    return pl.pallas_call(
        _att_exp_body,
        grid=(NB,),
        in_specs=[
            pl.BlockSpec((BE, EH_C), lambda i: (i, 0)),
            pl.BlockSpec((BE, 1), lambda i: (i, 0)),
            pl.BlockSpec((NB, 1, 72), lambda i: (0, 0, 0)),
        ],
        out_specs=pl.BlockSpec((BE, 48), lambda i: (i, 0)),
        out_shape=jax.ShapeDtypeStruct((E, 48), jnp.float32),
    )(uef, s, stats)


def _node_body(ni, residual, final, nf_ref, agg_ref, gamma_ref, beta_ref,
               w1_ref, b1_ref, w2_ref, b2_ref, wr_ref, br_ref, out_ref):
    nf = nf_ref[...]
    agg = agg_ref[...]
    N = float(nf.shape[0])
    K = ni + EH_C
    mu_n = jnp.sum(nf, axis=0)[None, :] / N
    ms_n = jnp.sum(nf * nf, axis=0)[None, :] / N
    mu_a = jnp.sum(agg, axis=0)[None, :] / N
    ms_a = jnp.sum(agg * agg, axis=0)[None, :] / N
    gamma = gamma_ref[...]
    beta = beta_ref[...]
    w1 = w1_ref[...]
    sg_n = jnp.sqrt((ms_n - mu_n * mu_n) + 1e-5)
    sg_a = jnp.sqrt((ms_a - mu_a * mu_a) + 1e-5)
    xn = (nf - mu_n) / sg_n * gamma[:, :ni] + beta[:, :ni]
    xa = (agg - mu_a) / sg_a * gamma[:, ni:K] + beta[:, ni:K]
    bias = b1_ref[...] + _dot(beta[:, K:], w1[K:, :])
    h = jnp.maximum(_dot(xn, w1[:ni, :]) + _dot(xa, w1[ni:K, :]) + bias, 0.0)
    unf = _dot(h, w2_ref[...]) + b2_ref[...]
    if residual:
        unf = unf + nf
    if final:
        pred = _dot(unf, wr_ref[...]) + br_ref[...]
        out_ref[...] = jnp.clip(pred, 0.0, 1.0)
    else:
        out_ref[...] = unf


def _node_mlp(nf, agg, nm, residual, final, reg):
    N, ni = nf.shape
    wr = reg["W"] if final else jnp.zeros((EH_C, 1), jnp.float32)
    br = reg["b"][None, :] if final else jnp.zeros((1, 1), jnp.float32)
    odim = 1 if final else EH_C
    return pl.pallas_call(
        partial(_node_body, ni, residual, final),
        out_shape=jax.ShapeDtypeStruct((N, odim), jnp.float32),
    )(nf, agg, nm["gamma"][None, :], nm["beta"][None, :], nm["W1"],
      nm["b1"][None, :], nm["W2"], nm["b2"][None, :], wr, br)


from jax.experimental.pallas import tpu_sc as plsc

SC_CH = 400
EPW = N_EDGES_C // 32
NCH = EPW // SC_CH
RPT = N_NODES_C // 16


def _sc_mesh():
    return plsc.VectorSubcoreMesh(core_axis_name="c", subcore_axis_name="s")


def _sc_gather_add(A, B, src, dst):
    @functools.partial(
        pl.kernel, mesh=_sc_mesh(),
        out_type=jax.ShapeDtypeStruct((N_EDGES_C, HID), jnp.float32),
        scratch_types=[
            pltpu.VMEM((SC_CH,), jnp.int32),
            pltpu.VMEM((SC_CH,), jnp.int32),
            pltpu.VMEM((SC_CH, HID), jnp.float32),
            pltpu.VMEM((SC_CH, HID), jnp.float32),
            pltpu.SemaphoreType.DMA,
            pltpu.SemaphoreType.DMA,
        ],
    )
    def k(a_hbm, b_hbm, src_hbm, dst_hbm, g_hbm,
          idx_s, idx_d, bufa, bufb, sema, semb):
        wid = lax.axis_index("s") * 2 + lax.axis_index("c")
        base = wid * EPW

        def chunk(c, _):
            off = base + c * SC_CH
            pltpu.sync_copy(src_hbm.at[pl.ds(off, SC_CH)], idx_s)
            pltpu.sync_copy(dst_hbm.at[pl.ds(off, SC_CH)], idx_d)
            ca = pltpu.async_copy(a_hbm.at[idx_s], bufa, sema)
            cb = pltpu.async_copy(b_hbm.at[idx_d], bufb, semb)
            ca.wait()
            cb.wait()

            def row(r, _):
                for j in range(HID // 16):
                    sl = pl.ds(j * 16, 16)
                    bufa[r, sl] = bufa[r, sl] + bufb[r, sl]
                return 0

            lax.fori_loop(0, SC_CH, row, 0)
            pltpu.sync_copy(bufa, g_hbm.at[pl.ds(off, SC_CH)])
            return 0

        lax.fori_loop(0, NCH, chunk, 0)

    return k(A, B, src, dst)


def _sc_scatter_width(width):
    @functools.partial(
        pl.kernel, mesh=_sc_mesh(),
        out_type=jax.ShapeDtypeStruct((2, N_NODES_C, width), jnp.float32),
        scratch_types=[
            pltpu.VMEM((SC_CH,), jnp.int32),
            pltpu.VMEM((SC_CH, width), jnp.float32),
            pltpu.VMEM((RPT, width), jnp.float32),
            pltpu.VMEM_SHARED((N_NODES_C, width), jnp.float32),
        ],
    )
    def k(x_hbm, idx_hbm, out_hbm, idx_v, pay_v, zb_v, shared):
        cid = lax.axis_index("c")
        sid = lax.axis_index("s")

        def zrow(r, _):
            for j in range(width // 16):
                zb_v[r, pl.ds(j * 16, 16)] = jnp.zeros((16,), jnp.float32)
            return 0

        lax.fori_loop(0, RPT, zrow, 0)
        pltpu.sync_copy(zb_v, shared.at[pl.ds(sid * RPT, RPT)])
        plsc.subcore_barrier()
        base = cid * (N_EDGES_C // 2) + sid * EPW

        def chunk(c, _):
            off = base + c * SC_CH
            pltpu.sync_copy(idx_hbm.at[pl.ds(off, SC_CH)], idx_v)
            pltpu.sync_copy(x_hbm.at[pl.ds(off, SC_CH)], pay_v)
            pltpu.sync_copy(pay_v, shared.at[idx_v], add=True)
            return 0

        lax.fori_loop(0, NCH, chunk, 0)
        plsc.subcore_barrier()
        pltpu.sync_copy(shared.at[pl.ds(sid * RPT, RPT)],
                        out_hbm.at[cid].at[pl.ds(sid * RPT, RPT)])

    return k


def _segsum_sc(x, idx, width):
    parts = _sc_scatter_width(width)(x, idx)
    return parts[0] + parts[1]


def _sc_degrees(src, dst):
    @functools.partial(
        pl.kernel, mesh=_sc_mesh(),
        out_type=[
            jax.ShapeDtypeStruct((2, N_NODES_C, 16), jnp.float32),
            jax.ShapeDtypeStruct((2, N_NODES_C, 16), jnp.float32),
        ],
        scratch_types=[
            pltpu.VMEM((SC_CH,), jnp.int32),
            pltpu.VMEM((SC_CH, 16), jnp.float32),
            pltpu.VMEM((RPT, 16), jnp.float32),
            pltpu.VMEM_SHARED((N_NODES_C, 16), jnp.float32),
            pltpu.VMEM_SHARED((N_NODES_C, 16), jnp.float32),
        ],
    )
    def k(src_hbm, dst_hbm, outs_hbm, outd_hbm,
          idx_v, pay_v, zb_v, shs, shd):
        cid = lax.axis_index("c")
        sid = lax.axis_index("s")
        one = jnp.where(lax.iota(jnp.int32, 16) == 0, 1.0, 0.0)

        def prow(r, _):
            pay_v[r, pl.ds(0, 16)] = one
            zb_v[jnp.minimum(r, RPT - 1), pl.ds(0, 16)] = jnp.zeros(
                (16,), jnp.float32)
            return 0

        lax.fori_loop(0, RPT, prow, 0)

        def prow2(r, _):
            pay_v[r, pl.ds(0, 16)] = one
            return 0

        lax.fori_loop(RPT, SC_CH, prow2, 0) if SC_CH > RPT else None
        pltpu.sync_copy(zb_v, shs.at[pl.ds(sid * RPT, RPT)])
        pltpu.sync_copy(zb_v, shd.at[pl.ds(sid * RPT, RPT)])
        plsc.subcore_barrier()
        base = cid * (N_EDGES_C // 2) + sid * EPW

        def chunk(c, _):
            off = base + c * SC_CH
            pltpu.sync_copy(src_hbm.at[pl.ds(off, SC_CH)], idx_v)
            pltpu.sync_copy(pay_v, shs.at[idx_v], add=True)
            pltpu.sync_copy(dst_hbm.at[pl.ds(off, SC_CH)], idx_v)
            pltpu.sync_copy(pay_v, shd.at[idx_v], add=True)
            return 0

        lax.fori_loop(0, NCH, chunk, 0)
        plsc.subcore_barrier()
        pltpu.sync_copy(shs.at[pl.ds(sid * RPT, RPT)],
                        outs_hbm.at[cid].at[pl.ds(sid * RPT, RPT)])
        pltpu.sync_copy(shd.at[pl.ds(sid * RPT, RPT)],
                        outd_hbm.at[cid].at[pl.ds(sid * RPT, RPT)])

    ps, pd = k(src, dst)
    degs = ps[0, :, :1] + ps[1, :, :1]
    degd = pd[0, :, :1] + pd[1, :, :1]
    return degs, degd


def _gather_add(A, B, src, dst):
    return A[src] + B[dst]


def _segsum(x, dst):
    return jax.ops.segment_sum(x, dst, num_segments=N_NODES_C)


def _degrees(src, dst):
    ones = jnp.ones((N_EDGES_C,), jnp.float32)
    degs = jax.ops.segment_sum(ones, src, num_segments=N_NODES_C)
    degd = jax.ops.segment_sum(ones, dst, num_segments=N_NODES_C)
    return degs[:, None], degd[:, None]


def kernel(nf, ef, gf, params, edge_index):
    src = edge_index[0].astype(jnp.int32)
    dst = edge_index[1].astype(jnp.int32)
    degs, degd = _degrees(src, dst)
    layers = params["layers"]
    L = len(layers)
    part = _ef_stats(ef)
    for i, lp in enumerate(layers):
        residual = i >= 1
        with_att = i == L - 1
        A, Bm, bias, mv = _edge_prep(nf, degs, degd, part, lp["em"],
                                     ef.shape[1])
        G = _gather_add(A, Bm, src, dst)
        att = lp.get("att", None)
        uef, stats, s = _edge_mlp(G, ef, mv, bias, lp["em"], att,
                                  residual, with_att)
        if with_att:
            ut = _att_exp(uef, s, stats)
            nd = _segsum(ut, dst)
            agg = nd[:, :EH_C] / (nd[:, EH_C:EH_C + 1] + 1e-16)
        else:
            agg = _segsum(uef, dst)
        nf = _node_mlp(nf, agg, lp["nm"], residual, i == L - 1,
                       params["reg"])
        ef = uef
        part = stats
    return nf


# speedup vs baseline: 4.0768x; 3.1121x over previous
import functools
from functools import partial

import jax
import jax.numpy as jnp
from jax import lax
from jax.experimental import pallas as pl
from jax.experimental.pallas import tpu as pltpu

N_NODES_C = 10000
N_EDGES_C = 320000
BE = 2560
NB = N_EDGES_C // BE
EH_C = 32
HID = 64
BF = jnp.bfloat16
F32 = jnp.float32


def _dot(a, b):
    return jnp.dot(a.astype(BF), b.astype(BF), preferred_element_type=F32)


def _ef_stats_body(ef_ref, out_ref):
    x = ef_ref[...]
    s = jnp.sum(x, axis=0)
    ss = jnp.sum(x * x, axis=0)
    out_ref[0, 0, :] = jnp.concatenate([s, ss])


def _ef_stats(ef):
    E, F = ef.shape
    return pl.pallas_call(
        _ef_stats_body,
        grid=(NB,),
        in_specs=[pl.BlockSpec((BE, F), lambda i: (i, 0))],
        out_specs=pl.BlockSpec((1, 1, 2 * F), lambda i: (i, 0, 0)),
        out_shape=jax.ShapeDtypeStruct((NB, 1, 2 * F), jnp.float32),
    )(ef)


def _edge_prep_body(ei, ni, nf_ref, deg_ref, part_ref,
                    gamma_ref, beta_ref, w1_ref, b1_ref,
                    t_ref, bias_ref, mv_ref):
    E = float(N_EDGES_C)
    nf = nf_ref[...]
    part = part_ref[...]
    ef_mu = jnp.sum(part[:, 0, :ei], axis=0)[None, :] / E
    ef_ms = jnp.sum(part[:, 0, ei:2 * ei], axis=0)[None, :] / E
    degs = deg_ref[0, :, 0:1] + deg_ref[1, :, 0:1]
    degd = deg_ref[0, :, 1:2] + deg_ref[1, :, 1:2]
    s_mu = jnp.sum(nf * degs, axis=0)[None, :] / E
    s_ms = jnp.sum(nf * nf * degs, axis=0)[None, :] / E
    d_mu = jnp.sum(nf * degd, axis=0)[None, :] / E
    d_ms = jnp.sum(nf * nf * degd, axis=0)[None, :] / E
    K = ei + 2 * ni
    gamma = gamma_ref[...]
    beta = beta_ref[...]
    w1 = w1_ref[...]
    sg_s = jnp.sqrt((s_ms - s_mu * s_mu) + 1e-5)
    sg_d = jnp.sqrt((d_ms - d_mu * d_mu) + 1e-5)
    xs = (nf - s_mu) / sg_s * gamma[:, ei:ei + ni] + beta[:, ei:ei + ni]
    xd = (nf - d_mu) / sg_d * gamma[:, ei + ni:K] + beta[:, ei + ni:K]
    t_ref[:, :HID] = _dot(xs, w1[ei:ei + ni, :])
    t_ref[:, HID:] = _dot(xd, w1[ei + ni:K, :])
    bias_ref[...] = b1_ref[...] + _dot(beta[:, K:], w1[K:, :])
    sg_ef = jnp.sqrt((ef_ms - ef_mu * ef_mu) + 1e-5)
    mv_ref[...] = jnp.concatenate([ef_mu, sg_ef], axis=1)


def _edge_prep(nf, deg, part, em, ei):
    N, ni = nf.shape
    return pl.pallas_call(
        partial(_edge_prep_body, ei, ni),
        out_shape=[
            jax.ShapeDtypeStruct((N, 2 * HID), jnp.float32),
            jax.ShapeDtypeStruct((1, HID), jnp.float32),
            jax.ShapeDtypeStruct((1, 2 * ei), jnp.float32),
        ],
    )(nf, deg, part, em["gamma"][None, :], em["beta"][None, :],
      em["W1"], em["b1"][None, :])


def _edge_body(residual, with_att, ei, g_ref, ef_ref, w1e_ref, gb_ref,
               mv_ref, bias_ref, w2_ref, b2_ref, att_ref,
               uef_ref, stat_ref, s_ref):
    g = g_ref[:, :HID]
    ef = ef_ref[:, :ei]
    mu = mv_ref[0, :ei][None, :]
    sg = mv_ref[0, ei:][None, :]
    gamma = gb_ref[0, :ei][None, :]
    beta = gb_ref[0, ei:][None, :]
    xe = (ef - mu) / sg * gamma + beta
    pre = g + _dot(xe, w1e_ref[...]) + bias_ref[...]
    h = jnp.maximum(pre, 0.0)
    uef = _dot(h, w2_ref[...]) + b2_ref[...]
    if residual:
        uef = uef + ef
    uef_ref[:, :EH_C] = uef
    uef_ref[:, EH_C:] = jnp.zeros((uef.shape[0], WID - EH_C), jnp.float32)
    s = jnp.sum(uef, axis=0)
    ss = jnp.sum(uef * uef, axis=0)
    if with_att:
        sa = _dot(uef, att_ref[...])
        s_ref[...] = sa
        mx = jnp.full((8,), jnp.max(sa), jnp.float32)
    else:
        mx = jnp.zeros((8,), jnp.float32)
    stat_ref[0, 0, :] = jnp.concatenate([s, ss, mx])


def _edge_mlp(G, ef, mv, bias, em, att, residual, with_att, ei):
    E = G.shape[0]
    att2 = att[:, None] if att is not None else jnp.zeros((EH_C, 1), jnp.float32)
    gb = jnp.concatenate([em["gamma"][None, :ei], em["beta"][None, :ei]],
                         axis=1)
    outs = pl.pallas_call(
        partial(_edge_body, residual, with_att, ei),
        grid=(NB,),
        in_specs=[
            pl.BlockSpec((BE, WID), lambda i: (i, 0)),
            pl.BlockSpec((BE, ef.shape[1]), lambda i: (i, 0)),
            pl.BlockSpec((ei, HID), lambda i: (0, 0)),
            pl.BlockSpec((1, 2 * ei), lambda i: (0, 0)),
            pl.BlockSpec((1, 2 * ei), lambda i: (0, 0)),
            pl.BlockSpec((1, HID), lambda i: (0, 0)),
            pl.BlockSpec((HID, EH_C), lambda i: (0, 0)),
            pl.BlockSpec((1, EH_C), lambda i: (0, 0)),
            pl.BlockSpec((EH_C, 1), lambda i: (0, 0)),
        ],
        out_specs=[
            pl.BlockSpec((BE, WID), lambda i: (i, 0)),
            pl.BlockSpec((1, 1, 72), lambda i: (i, 0, 0)),
            pl.BlockSpec((BE, 1), lambda i: (i, 0)),
        ],
        out_shape=[
            jax.ShapeDtypeStruct((E, WID), jnp.float32),
            jax.ShapeDtypeStruct((NB, 1, 72), jnp.float32),
            jax.ShapeDtypeStruct((E, 1), jnp.float32),
        ],
    )(G, ef, em["W1"][:ei], gb, mv, bias, em["W2"], em["b2"][None, :], att2)
    return outs


def _att_exp_body(uef_ref, s_ref, stat_ref, ut_ref):
    m = jnp.max(stat_ref[:, 0, 64:72])
    t = jnp.exp(s_ref[...] - m)
    ut_ref[:, :EH_C] = uef_ref[:, :EH_C] * t
    ut_ref[:, EH_C:] = jnp.concatenate(
        [t, jnp.zeros((t.shape[0], WID - EH_C - 1), jnp.float32)], axis=1)


def _att_exp(uef, s, stats):
    E = uef.shape[0]
    return pl.pallas_call(
        _att_exp_body,
        grid=(NB,),
        in_specs=[
            pl.BlockSpec((BE, WID), lambda i: (i, 0)),
            pl.BlockSpec((BE, 1), lambda i: (i, 0)),
            pl.BlockSpec((NB, 1, 72), lambda i: (0, 0, 0)),
        ],
        out_specs=pl.BlockSpec((BE, WID), lambda i: (i, 0)),
        out_shape=jax.ShapeDtypeStruct((E, WID), jnp.float32),
    )(uef, s, stats)


def _node_body(ni, residual, final, with_att, nf_ref, agg_ref, gamma_ref,
               beta_ref, w1_ref, b1_ref, w2_ref, b2_ref, wr_ref, br_ref,
               out_ref):
    nf = nf_ref[...]
    agg = agg_ref[0, :, :EH_C] + agg_ref[1, :, :EH_C]
    if with_att:
        den = agg_ref[0, :, EH_C:EH_C + 1] + agg_ref[1, :, EH_C:EH_C + 1]
        agg = agg / (den + 1e-16)
    N = float(nf.shape[0])
    K = ni + EH_C
    mu_n = jnp.sum(nf, axis=0)[None, :] / N
    ms_n = jnp.sum(nf * nf, axis=0)[None, :] / N
    mu_a = jnp.sum(agg, axis=0)[None, :] / N
    ms_a = jnp.sum(agg * agg, axis=0)[None, :] / N
    gamma = gamma_ref[...]
    beta = beta_ref[...]
    w1 = w1_ref[...]
    sg_n = jnp.sqrt((ms_n - mu_n * mu_n) + 1e-5)
    sg_a = jnp.sqrt((ms_a - mu_a * mu_a) + 1e-5)
    xn = (nf - mu_n) / sg_n * gamma[:, :ni] + beta[:, :ni]
    xa = (agg - mu_a) / sg_a * gamma[:, ni:K] + beta[:, ni:K]
    bias = b1_ref[...] + _dot(beta[:, K:], w1[K:, :])
    h = jnp.maximum(_dot(xn, w1[:ni, :]) + _dot(xa, w1[ni:K, :]) + bias, 0.0)
    unf = _dot(h, w2_ref[...]) + b2_ref[...]
    if residual:
        unf = unf + nf
    if final:
        pred = _dot(unf, wr_ref[...]) + br_ref[...]
        out_ref[...] = jnp.clip(pred, 0.0, 1.0)
    else:
        out_ref[...] = unf


def _node_mlp(nf, parts, nm, residual, final, reg, with_att):
    N, ni = nf.shape
    wr = reg["W"] if final else jnp.zeros((EH_C, 1), jnp.float32)
    br = reg["b"][None, :] if final else jnp.zeros((1, 1), jnp.float32)
    odim = 1 if final else EH_C
    return pl.pallas_call(
        partial(_node_body, ni, residual, final, with_att),
        out_shape=jax.ShapeDtypeStruct((N, odim), jnp.float32),
    )(nf, parts, nm["gamma"][None, :], nm["beta"][None, :], nm["W1"],
      nm["b1"][None, :], nm["W2"], nm["b2"][None, :], wr, br)


from jax.experimental.pallas import tpu_sc as plsc

SC_CH = 200
GA_CH = 400
DEG_CH = 80
EPW = N_EDGES_C // 32
STR = 640
LAST = N_NODES_C - 15 * STR
WID = 128


def _sc_mesh():
    return plsc.VectorSubcoreMesh(core_axis_name="c", subcore_axis_name="s")


def _stripe_copy(sid, do_full, do_last):
    @pl.when(sid < 15)
    def _():
        do_full()

    @pl.when(sid == 15)
    def _():
        do_last()


def _sc_gather_add(T, src, dst):
    @functools.partial(
        pl.kernel, mesh=_sc_mesh(),
        out_type=jax.ShapeDtypeStruct((N_EDGES_C, WID), jnp.float32),
        scratch_types=[
            pltpu.VMEM((GA_CH,), jnp.int32),
            pltpu.VMEM((GA_CH,), jnp.int32),
            pltpu.VMEM((GA_CH, WID), jnp.float32),
            pltpu.VMEM((GA_CH, WID), jnp.float32),
            pltpu.SemaphoreType.DMA,
            pltpu.SemaphoreType.DMA,
        ],
    )
    def k(t_hbm, src_hbm, dst_hbm, g_hbm,
          idx_s, idx_d, bufa, bufb, sema, semb):
        wid = lax.axis_index("s") * 2 + lax.axis_index("c")
        base = wid * EPW

        def chunk(c, _):
            off = base + c * GA_CH
            pltpu.sync_copy(src_hbm.at[pl.ds(off, GA_CH)], idx_s)
            pltpu.sync_copy(dst_hbm.at[pl.ds(off, GA_CH)], idx_d)
            ca = pltpu.async_copy(t_hbm.at[idx_s], bufa, sema)
            cb = pltpu.async_copy(t_hbm.at[idx_d], bufb, semb)
            ca.wait()
            cb.wait()

            def row(r, _):
                for j in range(HID // 16):
                    sl = pl.ds(j * 16, 16)
                    s2 = pl.ds(HID + j * 16, 16)
                    bufa[r, sl] = bufa[r, sl] + bufb[r, s2]
                return 0

            lax.fori_loop(0, GA_CH, row, 0)
            pltpu.sync_copy(bufa, g_hbm.at[pl.ds(off, GA_CH)])
            return 0

        lax.fori_loop(0, EPW // GA_CH, chunk, 0)

    return k(T, src, dst)


def _sc_scatter(x, idx, z):
    @functools.partial(
        pl.kernel, mesh=_sc_mesh(),
        out_type=jax.ShapeDtypeStruct((2, N_NODES_C, WID), jnp.float32),
        scratch_types=[
            pltpu.VMEM((SC_CH,), jnp.int32),
            pltpu.VMEM((SC_CH, WID), jnp.float32),
            pltpu.VMEM_SHARED((N_NODES_C, WID), jnp.float32),
        ],
    )
    def k(x_hbm, idx_hbm, z_hbm, out_hbm, idx_v, pay_v, shared):
        cid = lax.axis_index("c")
        sid = lax.axis_index("s")
        _stripe_copy(
            sid,
            lambda: pltpu.sync_copy(z_hbm.at[pl.ds(sid * STR, STR)],
                                    shared.at[pl.ds(sid * STR, STR)]),
            lambda: pltpu.sync_copy(z_hbm.at[pl.ds(15 * STR, LAST)],
                                    shared.at[pl.ds(15 * STR, LAST)]))
        plsc.subcore_barrier()
        base = cid * (N_EDGES_C // 2) + sid * EPW

        def chunk(c, _):
            off = base + c * SC_CH
            pltpu.sync_copy(idx_hbm.at[pl.ds(off, SC_CH)], idx_v)
            pltpu.sync_copy(x_hbm.at[pl.ds(off, SC_CH)], pay_v)
            pltpu.sync_copy(pay_v, shared.at[idx_v], add=True)
            return 0

        lax.fori_loop(0, EPW // SC_CH, chunk, 0)
        plsc.subcore_barrier()
        _stripe_copy(
            sid,
            lambda: pltpu.sync_copy(shared.at[pl.ds(sid * STR, STR)],
                                    out_hbm.at[cid].at[pl.ds(sid * STR, STR)]),
            lambda: pltpu.sync_copy(shared.at[pl.ds(15 * STR, LAST)],
                                    out_hbm.at[cid].at[pl.ds(15 * STR, LAST)]))

    return k(x, idx, z)


def _sc_degrees(src, dst, z):
    @functools.partial(
        pl.kernel, mesh=_sc_mesh(),
        out_type=jax.ShapeDtypeStruct((2, N_NODES_C, WID), jnp.float32),
        scratch_types=[
            pltpu.VMEM((DEG_CH,), jnp.int32),
            pltpu.VMEM((DEG_CH, WID), jnp.float32),
            pltpu.VMEM((DEG_CH, WID), jnp.float32),
            pltpu.VMEM_SHARED((N_NODES_C, WID), jnp.float32),
        ],
    )
    def k(src_hbm, dst_hbm, z_hbm, out_hbm, idx_v, pays_v, payd_v, sh):
        cid = lax.axis_index("c")
        sid = lax.axis_index("s")
        lane = lax.iota(jnp.int32, 16)
        ones = jnp.where(lane == 0, 1.0, 0.0)
        oned = jnp.where(lane == 1, 1.0, 0.0)
        zer = jnp.zeros((16,), jnp.float32)

        def prow(r, _):
            for j in range(WID // 16):
                pays_v[r, pl.ds(j * 16, 16)] = ones if j == 0 else zer
                payd_v[r, pl.ds(j * 16, 16)] = oned if j == 0 else zer
            return 0

        lax.fori_loop(0, DEG_CH, prow, 0)
        _stripe_copy(
            sid,
            lambda: pltpu.sync_copy(z_hbm.at[pl.ds(sid * STR, STR)],
                                    sh.at[pl.ds(sid * STR, STR)]),
            lambda: pltpu.sync_copy(z_hbm.at[pl.ds(15 * STR, LAST)],
                                    sh.at[pl.ds(15 * STR, LAST)]))
        plsc.subcore_barrier()
        base = cid * (N_EDGES_C // 2) + sid * EPW

        def chunk(c, _):
            off = base + c * DEG_CH
            pltpu.sync_copy(src_hbm.at[pl.ds(off, DEG_CH)], idx_v)
            pltpu.sync_copy(pays_v, sh.at[idx_v], add=True)
            pltpu.sync_copy(dst_hbm.at[pl.ds(off, DEG_CH)], idx_v)
            pltpu.sync_copy(payd_v, sh.at[idx_v], add=True)
            return 0

        lax.fori_loop(0, EPW // DEG_CH, chunk, 0)
        plsc.subcore_barrier()
        _stripe_copy(
            sid,
            lambda: pltpu.sync_copy(sh.at[pl.ds(sid * STR, STR)],
                                    out_hbm.at[cid].at[pl.ds(sid * STR, STR)]),
            lambda: pltpu.sync_copy(sh.at[pl.ds(15 * STR, LAST)],
                                    out_hbm.at[cid].at[pl.ds(15 * STR, LAST)]))

    return k(src, dst, z)


def _gather_add(A, B, src, dst):
    return A[src] + B[dst]


def _segsum(x, dst):
    return jax.ops.segment_sum(x, dst, num_segments=N_NODES_C)


def _degrees(src, dst):
    ones = jnp.ones((N_EDGES_C,), jnp.float32)
    degs = jax.ops.segment_sum(ones, src, num_segments=N_NODES_C)
    degd = jax.ops.segment_sum(ones, dst, num_segments=N_NODES_C)
    return degs[:, None], degd[:, None]


def kernel(nf, ef, gf, params, edge_index):
    src = edge_index[0].astype(jnp.int32)
    dst = edge_index[1].astype(jnp.int32)
    z = jnp.zeros((N_NODES_C, WID), jnp.float32)
    deg = _sc_degrees(src, dst, z)
    layers = params["layers"]
    L = len(layers)
    part = _ef_stats(ef)
    ei = ef.shape[1]
    for i, lp in enumerate(layers):
        residual = i >= 1
        with_att = i == L - 1
        T, bias, mv = _edge_prep(nf, deg, part, lp["em"], ei)
        G = _sc_gather_add(T, src, dst)
        att = lp.get("att", None)
        uef, stats, s = _edge_mlp(G, ef, mv, bias, lp["em"], att,
                                  residual, with_att, ei)
        if with_att:
            ut = _att_exp(uef, s, stats)
            parts = _sc_scatter(ut, dst, z)
        else:
            parts = _sc_scatter(uef, dst, z)
        nf = _node_mlp(nf, parts, lp["nm"], residual, i == L - 1,
                       params["reg"], with_att)
        ef = uef
        ei = EH_C
        part = stats
    return nf
